# baseline (speedup 1.0000x reference)
import jax
import jax.numpy as jnp
from jax.experimental import pallas as pl

_N = 100000
_D = 16
_K = 1024
_DA = _D + 3
_BN = 10240


def _assign_block(xt_ref, ca_ref, o_ref):
    xat = xt_ref[...]
    ca = ca_ref[...]
    d2 = jax.lax.dot_general(
        ca, xat, (((1,), (0,)), ((), ())),
        preferred_element_type=jnp.float32,
    )
    o_ref[0, 0, :] = jnp.argmin(d2, axis=0).astype(jnp.int32)


def kernel(X, centroids):
    grid = (_N + _BN - 1) // _BN
    b2 = jnp.sum(centroids * centroids, axis=1)
    mask = jnp.int32(-65536)

    def _trunc(v):
        return jax.lax.bitcast_convert_type(
            jax.lax.bitcast_convert_type(v, jnp.int32) & mask, jnp.float32)

    b2_hi = _trunc(b2)
    r = b2 - b2_hi
    b2_mid = _trunc(r)
    b2_lo = r - b2_mid
    ca = jnp.concatenate(
        [-2.0 * centroids, b2_hi[:, None], b2_mid[:, None], b2_lo[:, None]],
        axis=1,
    ).astype(jnp.bfloat16)
    xat = jnp.concatenate(
        [X.T.astype(jnp.bfloat16), jnp.ones((3, _N), jnp.bfloat16)], axis=0
    )
    out = pl.pallas_call(
        _assign_block,
        grid=(grid,),
        in_specs=[
            pl.BlockSpec((_DA, _BN), lambda i: (0, i)),
            pl.BlockSpec((_K, _DA), lambda i: (0, 0)),
        ],
        out_specs=pl.BlockSpec((1, 1, _BN), lambda i: (i, 0, 0)),
        out_shape=jax.ShapeDtypeStruct((grid, 1, _BN), jnp.int32),
    )(xat, ca)
    return out.reshape(grid * _BN)[:_N]
